# Initial kernel scaffold; baseline (speedup 1.0000x reference)
import functools

import jax
import jax.numpy as jnp
from jax import lax
from jax.experimental import pallas as pl
from jax.experimental.pallas import tpu as pltpu
from jax.experimental.pallas import tpu_sc as plsc

N = 10000
E = 320000
DF = 128
H = 20
C = 16

NC, NS = 2, 16
NW = NC * NS
PAD_N = 10240
TRASH = N
CHUNK = 128
CPW = 79
NCH = NW * CPW
E_PAD = NCH * CHUNK
RPT = PAD_N // NS

_mesh = lambda: plsc.VectorSubcoreMesh(core_axis_name="c", subcore_axis_name="s")


@functools.lru_cache(maxsize=None)
def _deg_kernel():
    @functools.partial(
        pl.kernel, mesh=_mesh(),
        out_type=jax.ShapeDtypeStruct((NC, PAD_N), jnp.float32),
        scratch_types=[
            pltpu.VMEM((CHUNK,), jnp.int32),
            pltpu.VMEM((CHUNK,), jnp.float32),
            pltpu.VMEM_SHARED((PAD_N,), jnp.float32),
            pltpu.SemaphoreType.DMA,
        ],
    )
    def k(dstc_hbm, zeros_hbm, out_hbm, dst_v, ones_v, deg_sh, sem):
        c = lax.axis_index("c")
        s = lax.axis_index("s")
        wid = c * NS + s
        for i in range(CHUNK // 16):
            ones_v[pl.ds(i * 16, 16)] = jnp.ones((16,), jnp.float32)
        r0 = s * RPT
        pltpu.sync_copy(zeros_hbm.at[pl.ds(r0, RPT)], deg_sh.at[pl.ds(r0, RPT)])
        plsc.subcore_barrier()

        def body(j, carry):
            cid = wid * CPW + j
            pltpu.sync_copy(dstc_hbm.at[cid], dst_v)
            pltpu.sync_copy(ones_v, deg_sh.at[dst_v], add=True)
            return carry

        lax.fori_loop(0, CPW, body, 0)
        plsc.subcore_barrier()
        pltpu.sync_copy(deg_sh.at[pl.ds(r0, RPT)], out_hbm.at[c].at[pl.ds(r0, RPT)])

    return k


@functools.lru_cache(maxsize=None)
def _agg_kernel(d):
    @functools.partial(
        pl.kernel, mesh=_mesh(),
        out_type=jax.ShapeDtypeStruct((NC, PAD_N, d), jnp.float32),
        scratch_types=[
            pltpu.VMEM((CHUNK,), jnp.int32),
            pltpu.VMEM((CHUNK,), jnp.int32),
            pltpu.VMEM((CHUNK, d), jnp.float32),
            pltpu.VMEM_SHARED((PAD_N, d), jnp.float32),
            pltpu.SemaphoreType.DMA,
        ],
    )
    def k(hp_hbm, srcc_hbm, dstc_hbm, zeros_hbm, out_hbm,
          src_v, dst_v, rows_v, agg_sh, sem):
        c = lax.axis_index("c")
        s = lax.axis_index("s")
        wid = c * NS + s
        r0 = s * RPT
        pltpu.sync_copy(zeros_hbm.at[pl.ds(r0, RPT)], agg_sh.at[pl.ds(r0, RPT)])
        plsc.subcore_barrier()

        def body(j, carry):
            cid = wid * CPW + j
            pltpu.sync_copy(srcc_hbm.at[cid], src_v)
            pltpu.async_copy(hp_hbm.at[src_v], rows_v, sem).wait()
            pltpu.sync_copy(dstc_hbm.at[cid], dst_v)
            pltpu.sync_copy(rows_v, agg_sh.at[dst_v], add=True)
            return carry

        lax.fori_loop(0, CPW, body, 0)
        plsc.subcore_barrier()
        pltpu.sync_copy(agg_sh.at[pl.ds(r0, RPT)],
                        out_hbm.at[c].at[pl.ds(r0, RPT)])

    return k


def _tc_scale_matmul(x_p, W1, degt):
    BN = 1024

    def body(x_ref, w_ref, deg_ref, hp_ref, dinv_ref):
        deg = deg_ref[:, 0:1] + deg_ref[:, 1:2] + 1.0
        dinv = lax.rsqrt(deg)
        h = jnp.dot(x_ref[...], w_ref[...], preferred_element_type=jnp.float32)
        hp_ref[...] = h * dinv
        dinv_ref[...] = dinv

    return pl.pallas_call(
        body,
        grid=(PAD_N // BN,),
        in_specs=[
            pl.BlockSpec((BN, DF), lambda i: (i, 0)),
            pl.BlockSpec((DF, H), lambda i: (0, 0)),
            pl.BlockSpec((BN, NC), lambda i: (i, 0)),
        ],
        out_specs=[
            pl.BlockSpec((BN, H), lambda i: (i, 0)),
            pl.BlockSpec((BN, 1), lambda i: (i, 0)),
        ],
        out_shape=[
            jax.ShapeDtypeStruct((PAD_N, H), jnp.float32),
            jax.ShapeDtypeStruct((PAD_N, 1), jnp.float32),
        ],
    )(x_p, W1, degt)


def _tc_mid(aggp, hp, dinv, b1, W2):
    BN = 1024

    def body(a_ref, hp_ref, dinv_ref, b1_ref, w2_ref, out_ref):
        p = a_ref[0] + a_ref[1] + hp_ref[...]
        h1 = jnp.maximum(dinv_ref[...] * p + b1_ref[...], 0.0)
        h2 = jnp.dot(h1, w2_ref[...], preferred_element_type=jnp.float32)
        out_ref[...] = h2 * dinv_ref[...]

    return pl.pallas_call(
        body,
        grid=(PAD_N // BN,),
        in_specs=[
            pl.BlockSpec((NC, BN, H), lambda i: (0, i, 0)),
            pl.BlockSpec((BN, H), lambda i: (i, 0)),
            pl.BlockSpec((BN, 1), lambda i: (i, 0)),
            pl.BlockSpec((1, H), lambda i: (0, 0)),
            pl.BlockSpec((H, C), lambda i: (0, 0)),
        ],
        out_specs=pl.BlockSpec((BN, C), lambda i: (i, 0)),
        out_shape=jax.ShapeDtypeStruct((PAD_N, C), jnp.float32),
    )(aggp, hp, dinv, b1, W2)


def _tc_final(aggp, h2p, dinv, b2):
    BN = 1024

    def body(a_ref, hp_ref, dinv_ref, b2_ref, out_ref):
        z = dinv_ref[...] * (a_ref[0] + a_ref[1] + hp_ref[...]) + b2_ref[...]
        m = jnp.max(z, axis=1, keepdims=True)
        e = jnp.exp(z - m)
        out_ref[...] = (z - m) - jnp.log(jnp.sum(e, axis=1, keepdims=True))

    return pl.pallas_call(
        body,
        grid=(PAD_N // BN,),
        in_specs=[
            pl.BlockSpec((NC, BN, C), lambda i: (0, i, 0)),
            pl.BlockSpec((BN, C), lambda i: (i, 0)),
            pl.BlockSpec((BN, 1), lambda i: (i, 0)),
            pl.BlockSpec((1, C), lambda i: (0, 0)),
        ],
        out_specs=pl.BlockSpec((BN, C), lambda i: (i, 0)),
        out_shape=jax.ShapeDtypeStruct((PAD_N, C), jnp.float32),
    )(aggp, h2p, dinv, b2)


def kernel(x, edge_index, W1, b1, W2, b2):
    ei = edge_index.astype(jnp.int32)
    pad_e = E_PAD - E
    srcc = jnp.concatenate(
        [ei[0], jnp.zeros((pad_e,), jnp.int32)]).reshape(NCH, CHUNK)
    dstc = jnp.concatenate(
        [ei[1], jnp.full((pad_e,), TRASH, jnp.int32)]).reshape(NCH, CHUNK)
    x_p = jnp.pad(x, ((0, PAD_N - N), (0, 0)))
    z1 = jnp.zeros((PAD_N,), jnp.float32)
    zh = jnp.zeros((PAD_N, H), jnp.float32)
    zc = jnp.zeros((PAD_N, C), jnp.float32)

    degp = _deg_kernel()(dstc, z1)
    degt = degp.T
    hp, dinv = _tc_scale_matmul(x_p, W1, degt)
    agg1 = _agg_kernel(H)(hp, srcc, dstc, zh)
    h2p = _tc_mid(agg1, hp, dinv, b1.reshape(1, H), W2)
    agg2 = _agg_kernel(C)(h2p, srcc, dstc, zc)
    out = _tc_final(agg2, h2p, dinv, b2.reshape(1, C))
    return out[:N]


# speedup vs baseline: 10.3359x; 10.3359x over previous
import functools

import jax
import jax.numpy as jnp
from jax import lax
from jax.experimental import pallas as pl
from jax.experimental.pallas import tpu as pltpu
from jax.experimental.pallas import tpu_sc as plsc

N = 10000
E = 320000
DF = 128
H = 20
C = 16
W = 128

NC, NS = 2, 16
NW = NC * NS
PAD_N = 10240
TRASH = N
CHUNK = 128
CPW = 79
NCH = NW * CPW
E_PAD = NCH * CHUNK
RPT = PAD_N // NS

_mesh = lambda: plsc.VectorSubcoreMesh(core_axis_name="c", subcore_axis_name="s")
_sc_params = lambda: pltpu.CompilerParams(use_tc_tiling_on_sc=False)


@functools.lru_cache(maxsize=None)
def _deg_kernel():
    @functools.partial(
        pl.kernel, mesh=_mesh(), compiler_params=_sc_params(),
        out_type=jax.ShapeDtypeStruct((NC, PAD_N), jnp.float32),
        scratch_types=[
            pltpu.VMEM((CHUNK,), jnp.int32),
            pltpu.VMEM((CHUNK,), jnp.float32),
            pltpu.VMEM_SHARED((PAD_N,), jnp.float32),
            pltpu.SemaphoreType.DMA,
        ],
    )
    def k(dstc_hbm, zeros_hbm, out_hbm, dst_v, ones_v, deg_sh, sem):
        c = lax.axis_index("c")
        s = lax.axis_index("s")
        wid = c * NS + s
        for i in range(CHUNK // 16):
            ones_v[pl.ds(i * 16, 16)] = jnp.ones((16,), jnp.float32)
        r0 = s * RPT
        pltpu.sync_copy(zeros_hbm.at[pl.ds(r0, RPT)], deg_sh.at[pl.ds(r0, RPT)])
        plsc.subcore_barrier()

        def body(j, carry):
            cid = wid * CPW + j
            pltpu.sync_copy(dstc_hbm.at[cid], dst_v)
            pltpu.sync_copy(ones_v, deg_sh.at[dst_v], add=True)
            return carry

        lax.fori_loop(0, CPW, body, 0)
        plsc.subcore_barrier()
        pltpu.sync_copy(deg_sh.at[pl.ds(r0, RPT)], out_hbm.at[c].at[pl.ds(r0, RPT)])

    return k


@functools.lru_cache(maxsize=None)
def _agg_kernel():
    @functools.partial(
        pl.kernel, mesh=_mesh(), compiler_params=_sc_params(),
        out_type=jax.ShapeDtypeStruct((NC, PAD_N, W), jnp.float32),
        scratch_types=[
            pltpu.VMEM((CHUNK,), jnp.int32),
            pltpu.VMEM((CHUNK,), jnp.int32),
            pltpu.VMEM((CHUNK, W), jnp.float32),
            pltpu.VMEM_SHARED((PAD_N, W), jnp.float32),
            pltpu.SemaphoreType.DMA,
        ],
    )
    def k(hp_hbm, srcc_hbm, dstc_hbm, zeros_hbm, out_hbm,
          src_v, dst_v, rows_v, agg_sh, sem):
        c = lax.axis_index("c")
        s = lax.axis_index("s")
        wid = c * NS + s
        r0 = s * RPT
        pltpu.sync_copy(zeros_hbm.at[pl.ds(r0, RPT)], agg_sh.at[pl.ds(r0, RPT)])
        plsc.subcore_barrier()

        def body(j, carry):
            cid = wid * CPW + j
            pltpu.sync_copy(srcc_hbm.at[cid], src_v)
            pltpu.async_copy(hp_hbm.at[src_v], rows_v, sem).wait()
            pltpu.sync_copy(dstc_hbm.at[cid], dst_v)
            pltpu.sync_copy(rows_v, agg_sh.at[dst_v], add=True)
            return carry

        lax.fori_loop(0, CPW, body, 0)
        plsc.subcore_barrier()
        pltpu.sync_copy(agg_sh.at[pl.ds(r0, RPT)],
                        out_hbm.at[c].at[pl.ds(r0, RPT)])

    return k


def _tc_scale_matmul(x_p, W1, degt):
    BN = 1024

    def body(x_ref, w_ref, deg_ref, hp_ref, dinv_ref):
        deg = deg_ref[:, 0:1] + deg_ref[:, 1:2] + 1.0
        dinv = lax.rsqrt(deg)
        h = jnp.dot(x_ref[...], w_ref[...], preferred_element_type=jnp.float32)
        hp_ref[...] = jnp.pad(h * dinv, ((0, 0), (0, W - H)))
        dinv_ref[...] = dinv

    return pl.pallas_call(
        body,
        grid=(PAD_N // BN,),
        in_specs=[
            pl.BlockSpec((BN, DF), lambda i: (i, 0)),
            pl.BlockSpec((DF, H), lambda i: (0, 0)),
            pl.BlockSpec((BN, NC), lambda i: (i, 0)),
        ],
        out_specs=[
            pl.BlockSpec((BN, W), lambda i: (i, 0)),
            pl.BlockSpec((BN, 1), lambda i: (i, 0)),
        ],
        out_shape=[
            jax.ShapeDtypeStruct((PAD_N, W), jnp.float32),
            jax.ShapeDtypeStruct((PAD_N, 1), jnp.float32),
        ],
    )(x_p, W1, degt)


def _tc_mid(aggp, hp, dinv, b1, W2):
    BN = 1024

    def body(a_ref, hp_ref, dinv_ref, b1_ref, w2_ref, out_ref):
        p = (a_ref[0] + a_ref[1] + hp_ref[...])[:, :H]
        h1 = jnp.maximum(dinv_ref[...] * p + b1_ref[...], 0.0)
        h2 = jnp.dot(h1, w2_ref[...], preferred_element_type=jnp.float32)
        out_ref[...] = jnp.pad(h2 * dinv_ref[...], ((0, 0), (0, W - C)))

    return pl.pallas_call(
        body,
        grid=(PAD_N // BN,),
        in_specs=[
            pl.BlockSpec((NC, BN, W), lambda i: (0, i, 0)),
            pl.BlockSpec((BN, W), lambda i: (i, 0)),
            pl.BlockSpec((BN, 1), lambda i: (i, 0)),
            pl.BlockSpec((1, H), lambda i: (0, 0)),
            pl.BlockSpec((H, C), lambda i: (0, 0)),
        ],
        out_specs=pl.BlockSpec((BN, W), lambda i: (i, 0)),
        out_shape=jax.ShapeDtypeStruct((PAD_N, W), jnp.float32),
    )(aggp, hp, dinv, b1, W2)


def _tc_final(aggp, h2p, dinv, b2):
    BN = 1024

    def body(a_ref, hp_ref, dinv_ref, b2_ref, out_ref):
        q = (a_ref[0] + a_ref[1] + hp_ref[...])[:, :C]
        z = dinv_ref[...] * q + b2_ref[...]
        m = jnp.max(z, axis=1, keepdims=True)
        e = jnp.exp(z - m)
        out_ref[...] = (z - m) - jnp.log(jnp.sum(e, axis=1, keepdims=True))

    return pl.pallas_call(
        body,
        grid=(PAD_N // BN,),
        in_specs=[
            pl.BlockSpec((NC, BN, W), lambda i: (0, i, 0)),
            pl.BlockSpec((BN, W), lambda i: (i, 0)),
            pl.BlockSpec((BN, 1), lambda i: (i, 0)),
            pl.BlockSpec((1, C), lambda i: (0, 0)),
        ],
        out_specs=pl.BlockSpec((BN, C), lambda i: (i, 0)),
        out_shape=jax.ShapeDtypeStruct((PAD_N, C), jnp.float32),
    )(aggp, h2p, dinv, b2)


def kernel(x, edge_index, W1, b1, W2, b2):
    ei = edge_index.astype(jnp.int32)
    pad_e = E_PAD - E
    srcc = jnp.concatenate(
        [ei[0], jnp.zeros((pad_e,), jnp.int32)]).reshape(NCH, CHUNK)
    dstc = jnp.concatenate(
        [ei[1], jnp.full((pad_e,), TRASH, jnp.int32)]).reshape(NCH, CHUNK)
    x_p = jnp.pad(x, ((0, PAD_N - N), (0, 0)))
    z1 = jnp.zeros((PAD_N,), jnp.float32)
    zw = jnp.zeros((PAD_N, W), jnp.float32)

    degp = _deg_kernel()(dstc, z1)
    degt = degp.T
    hp, dinv = _tc_scale_matmul(x_p, W1, degt)
    agg1 = _agg_kernel()(hp, srcc, dstc, zw)
    h2p = _tc_mid(agg1, hp, dinv, b1.reshape(1, H), W2)
    agg2 = _agg_kernel()(h2p, srcc, dstc, zw)
    out = _tc_final(agg2, h2p, dinv, b2.reshape(1, C))
    return out[:N]
